# baseline (speedup 1.0000x reference)
import functools

import jax
import jax.numpy as jnp
from jax import lax
from jax.experimental import pallas as pl
from jax.experimental.pallas import tpu as pltpu
from jax.experimental.pallas import tpu_sc as plsc

N = 10000
E = 320000
D = 128

NC = 2
NS = 16
NW = NC * NS
B = 128
NB_W = 79
E_PAD = NW * NB_W * B
ACC_N = 10112
ROWS_T = ACC_N // NS

_mesh = functools.partial(
    plsc.VectorSubcoreMesh, core_axis_name="c", subcore_axis_name="s")


def _deg_body(dst1d, out_hbm, idx_v, acc_v, *, nb):
    cid = lax.axis_index("c")
    sid = lax.axis_index("s")
    wid = sid * NC + cid
    nrow = ACC_N // B

    def zrow(i, _):
        def zcol(j, _):
            acc_v[i, pl.ds(j * 16, 16)] = jnp.zeros((16,), jnp.float32)
            return 0
        lax.fori_loop(0, B // 16, zcol, 0)
        return 0

    lax.fori_loop(0, nrow, zrow, 0)

    ones = jnp.ones((16,), jnp.float32)
    seven = jnp.full((16,), 7, jnp.int32)
    mask7 = jnp.full((16,), B - 1, jnp.int32)

    pltpu.sync_copy(dst1d.at[pl.ds(wid * nb * B, nb * B)], idx_v)

    def inner(j, _):
        idx = idx_v[pl.ds(j * 16, 16)]
        hi = lax.shift_right_logical(idx, seven)
        lo = lax.bitwise_and(idx, mask7)
        plsc.addupdate_scatter(acc_v, [hi, lo], ones)
        return 0

    lax.fori_loop(0, nb * B // 16, inner, 0)
    pltpu.sync_copy(acc_v, out_hbm.at[wid])


def _deg_call(dst1d):
    nb = dst1d.shape[0] // (NW * B)
    body = functools.partial(_deg_body, nb=nb)
    return pl.kernel(
        body,
        out_type=jax.ShapeDtypeStruct((NW, ACC_N // B, B), jnp.float32),
        mesh=_mesh(),
        scratch_types=[
            pltpu.VMEM((NB_W * B,), jnp.int32),
            pltpu.VMEM((ACC_N // B, B), jnp.float32),
        ],
        compiler_params=pltpu.CompilerParams(needs_layout_passes=False),
    )(dst1d)


NB0 = 146
NB1 = 2 * NB_W - NB0


_SLOTS = 2


def _agg_body(table, src2d, dst2d, z128, out_hbm,
              sidx_v, didx_v, bufs_v, acc_sh, *sems, nbr):
    cid = lax.axis_index("c")
    sid = lax.axis_index("s")
    r0 = sid * ROWS_T
    isem = sems[:2 * _SLOTS]
    gsem = sems[2 * _SLOTS:3 * _SLOTS]
    ssem = sems[3 * _SLOTS:]
    nb = jnp.where(cid == 0, NB0, NB1)
    base_row = sid * (NB0 + NB1) + cid * NB0
    pltpu.sync_copy(z128.at[pl.ds(r0, ROWS_T)], acc_sh.at[pl.ds(r0, ROWS_T)])
    plsc.subcore_barrier()

    def buf(k):
        return bufs_v.at[pl.ds(k * B, B)]

    def fire_idx(b, k):
        pltpu.async_copy(src2d.at[pl.ds(base_row + b, 1)],
                         sidx_v.at[pl.ds(k, 1)], isem[k])
        pltpu.async_copy(dst2d.at[pl.ds(base_row + b, 1)],
                         didx_v.at[pl.ds(k, 1)], isem[k])

    def wait_idx(k):
        pltpu.make_async_copy(src2d.at[pl.ds(0, 1)],
                              sidx_v.at[pl.ds(k, 1)], isem[k]).wait()
        pltpu.make_async_copy(dst2d.at[pl.ds(0, 1)],
                              didx_v.at[pl.ds(k, 1)], isem[k]).wait()

    def wait_scatter(k):
        pltpu.make_async_copy(z128.at[pl.ds(0, B)], buf(k), ssem[k]).wait()

    def wait_gather(k):
        pltpu.make_async_copy(z128.at[pl.ds(0, B)], buf(k), gsem[k]).wait()

    fire_idx(0, 0)

    def macro(m, _):
        for k in range(2 * _SLOTS):
            b = m * (2 * _SLOTS) + k
            q = k % _SLOTS
            kn = (k + 1) % (2 * _SLOTS)
            qs = (k - 1) % _SLOTS
            ki = (k - 1) % (2 * _SLOTS)

            @pl.when(b + 1 < nb)
            def _():
                fire_idx(b + 1, kn)

            @pl.when(b < nb)
            def _():
                @pl.when(b >= _SLOTS)
                def _():
                    wait_scatter(q)
                wait_idx(k)
                pltpu.async_copy(table.at[sidx_v.at[k]], buf(q), gsem[q])

            bs = b - 1

            @pl.when(jnp.logical_and(bs >= 0, bs < nb))
            def _():
                wait_gather(qs)
                pltpu.async_copy(buf(qs), acc_sh.at[didx_v.at[ki]],
                                 ssem[qs], add=True)
        return 0

    lax.fori_loop(0, (nb + 2 * _SLOTS) // (2 * _SLOTS), macro, 0)
    for k in range(_SLOTS):
        wait_scatter(k)
    plsc.subcore_barrier()
    pltpu.sync_copy(acc_sh.at[pl.ds(r0, ROWS_T)],
                    out_hbm.at[cid, pl.ds(r0, ROWS_T)])


def _agg_call(table, src2d, dst2d, z128):
    body = functools.partial(_agg_body, nbr=0)
    return pl.kernel(
        body,
        out_type=jax.ShapeDtypeStruct((NC, ACC_N, D), jnp.float32),
        mesh=_mesh(),
        scratch_types=[
            pltpu.VMEM((2 * _SLOTS, B), jnp.int32),
            pltpu.VMEM((2 * _SLOTS, B), jnp.int32),
            pltpu.VMEM((_SLOTS * B, D), jnp.float32),
            pltpu.VMEM_SHARED((ACC_N, D), jnp.float32),
        ] + [pltpu.SemaphoreType.DMA] * (4 * _SLOTS),
    )(table, src2d, dst2d, z128)


_R = 1000


def _dis_body(dp_ref, o_ref):
    deg = 1.0 + jnp.sum(dp_ref[...], axis=0)
    o_ref[...] = lax.rsqrt(deg)


def _dis_call(dp):
    return pl.pallas_call(
        _dis_body,
        out_shape=jax.ShapeDtypeStruct((ACC_N // B, B), jnp.float32),
    )(dp)


def _t1_body(x_ref, fm_ref, w1_ref, dis_ref, o_ref):
    xm = x_ref[...] * jax.nn.sigmoid(fm_ref[...])
    o_ref[...] = jnp.dot(xm, w1_ref[...],
                         preferred_element_type=jnp.float32) * dis_ref[...]


def _t2_body(p_ref, y1_ref, dis_ref, b1_ref, w2_ref, o_ref):
    dis = dis_ref[...]
    s = p_ref[0] + p_ref[1]
    h = jax.nn.relu(dis * (s + y1_ref[...]) + b1_ref[...])
    o_ref[...] = jnp.dot(h, w2_ref[...],
                         preferred_element_type=jnp.float32) * dis


def _t3_body(p_ref, y2_ref, dis_ref, b2_ref, wc_ref, bc_ref, o_ref):
    dis = dis_ref[...]
    z = dis * (p_ref[0] + p_ref[1] + y2_ref[...]) + b2_ref[...]
    o_ref[...] = jnp.dot(z, wc_ref[...],
                         preferred_element_type=jnp.float32) + bc_ref[...]


def _row_specs():
    full = pl.BlockSpec((1, D), lambda i: (0, 0))
    mat = pl.BlockSpec((D, D), lambda i: (0, 0))
    rows = pl.BlockSpec((_R, D), lambda i: (i, 0))
    dis = pl.BlockSpec((_R, 1), lambda i: (i, 0))
    pair = pl.BlockSpec((2, _R, D), lambda i: (0, i, 0))
    return full, mat, rows, dis, pair


def _t1_call(x, fm, w1, dis):
    full, mat, rows, dis_s, _ = _row_specs()
    return pl.pallas_call(
        _t1_body,
        grid=(N // _R,),
        in_specs=[rows, full, mat, dis_s],
        out_specs=rows,
        out_shape=jax.ShapeDtypeStruct((N, D), jnp.float32),
    )(x, fm, w1, dis)


def _t2_call(p, y1, dis, b1, w2):
    full, mat, rows, dis_s, pair = _row_specs()
    return pl.pallas_call(
        _t2_body,
        grid=(N // _R,),
        in_specs=[pair, rows, dis_s, full, mat],
        out_specs=rows,
        out_shape=jax.ShapeDtypeStruct((N, D), jnp.float32),
    )(p, y1, dis, b1, w2)


def _t3_call(p, y2, dis, b2, wc, bc):
    full, mat, rows, dis_s, pair = _row_specs()
    return pl.pallas_call(
        _t3_body,
        grid=(N // _R,),
        in_specs=[pair, rows, dis_s, full, mat, full],
        out_specs=rows,
        out_shape=jax.ShapeDtypeStruct((N, D), jnp.float32),
    )(p, y2, dis, b2, wc, bc)


def kernel(x, edge_index, feature_mask, W1, b1, W2, b2, Wc, bc):
    src = edge_index[0]
    dst = edge_index[1]
    pad = E_PAD - E
    src_p = jnp.concatenate([src, jnp.zeros((pad,), jnp.int32)])
    dst_p = jnp.concatenate([dst, jnp.full((pad,), N, jnp.int32)])
    src2d = src_p.reshape(E_PAD // B, B)
    dst2d = dst_p.reshape(E_PAD // B, B)

    z128 = jnp.zeros((ACC_N, D), jnp.float32)

    fm = feature_mask.reshape(1, D)
    b1r = b1.reshape(1, D)
    b2r = b2.reshape(1, D)
    wc_pad = jnp.pad(Wc, ((0, 0), (0, D - Wc.shape[1])))
    bc_pad = jnp.pad(bc, (0, D - bc.shape[0])).reshape(1, D)

    dp = _deg_call(dst_p)
    dis = _dis_call(dp).reshape(ACC_N)[:N].reshape(N, 1)

    y1 = _t1_call(x, fm, W1, dis)
    p1 = _agg_call(y1, src2d, dst2d, z128)[:, :N]
    y2 = _t2_call(p1, y1, dis, b1r, W2)
    p2 = _agg_call(y2, src2d, dst2d, z128)[:, :N]
    out = _t3_call(p2, y2, dis, b2r, wc_pad, bc_pad)
    return out[:, :Wc.shape[1]]


# speedup vs baseline: 1.1674x; 1.1674x over previous
import functools

import jax
import jax.numpy as jnp
from jax import lax
from jax.experimental import pallas as pl
from jax.experimental.pallas import tpu as pltpu
from jax.experimental.pallas import tpu_sc as plsc

N = 10000
E = 320000
D = 128

NC = 2
NS = 16
NW = NC * NS
B = 128
NB_W = 79
E_PAD = NW * NB_W * B
ACC_N = 10112
ROWS_T = ACC_N // NS

_mesh = functools.partial(
    plsc.VectorSubcoreMesh, core_axis_name="c", subcore_axis_name="s")


def _deg_body(dst1d, out_hbm, idx_v, acc_v, *, nb):
    cid = lax.axis_index("c")
    sid = lax.axis_index("s")
    wid = sid * NC + cid
    nrow = ACC_N // B

    def zrow(i, _):
        def zcol(j, _):
            acc_v[i, pl.ds(j * 16, 16)] = jnp.zeros((16,), jnp.float32)
            return 0
        lax.fori_loop(0, B // 16, zcol, 0)
        return 0

    lax.fori_loop(0, nrow, zrow, 0)

    ones = jnp.ones((16,), jnp.float32)
    seven = jnp.full((16,), 7, jnp.int32)
    mask7 = jnp.full((16,), B - 1, jnp.int32)

    pltpu.sync_copy(dst1d.at[pl.ds(wid * nb * B, nb * B)], idx_v)

    def inner(j, _):
        idx = idx_v[pl.ds(j * 16, 16)]
        hi = lax.shift_right_logical(idx, seven)
        lo = lax.bitwise_and(idx, mask7)
        plsc.addupdate_scatter(acc_v, [hi, lo], ones)
        return 0

    lax.fori_loop(0, nb * B // 16, inner, 0)
    pltpu.sync_copy(acc_v, out_hbm.at[wid])


def _deg_call(dst1d):
    nb = dst1d.shape[0] // (NW * B)
    body = functools.partial(_deg_body, nb=nb)
    return pl.kernel(
        body,
        out_type=jax.ShapeDtypeStruct((NW, ACC_N // B, B), jnp.float32),
        mesh=_mesh(),
        scratch_types=[
            pltpu.VMEM((NB_W * B,), jnp.int32),
            pltpu.VMEM((ACC_N // B, B), jnp.float32),
        ],
        compiler_params=pltpu.CompilerParams(needs_layout_passes=False),
    )(dst1d)


NB0 = 138
NB1 = 2 * NB_W - NB0


_SLOTS = 2


def _agg_body(table, src2d, dst2d, z128, out_hbm,
              sidx_v, didx_v, bufs_v, acc_sh, *sems, nbr):
    cid = lax.axis_index("c")
    sid = lax.axis_index("s")
    r0 = sid * ROWS_T
    isem = sems[:2 * _SLOTS]
    gsem = sems[2 * _SLOTS:3 * _SLOTS]
    ssem = sems[3 * _SLOTS:]
    nb = jnp.where(cid == 0, NB0, NB1)
    base_row = sid * (NB0 + NB1) + cid * NB0
    pltpu.sync_copy(z128.at[pl.ds(r0, ROWS_T)], acc_sh.at[pl.ds(r0, ROWS_T)])
    plsc.subcore_barrier()

    def buf(k):
        return bufs_v.at[pl.ds(k * B, B)]

    def fire_idx(b, k):
        pltpu.async_copy(src2d.at[pl.ds(base_row + b, 1)],
                         sidx_v.at[pl.ds(k, 1)], isem[k])
        pltpu.async_copy(dst2d.at[pl.ds(base_row + b, 1)],
                         didx_v.at[pl.ds(k, 1)], isem[k])

    def wait_idx(k):
        pltpu.make_async_copy(src2d.at[pl.ds(0, 1)],
                              sidx_v.at[pl.ds(k, 1)], isem[k]).wait()
        pltpu.make_async_copy(dst2d.at[pl.ds(0, 1)],
                              didx_v.at[pl.ds(k, 1)], isem[k]).wait()

    def wait_scatter(k):
        pltpu.make_async_copy(z128.at[pl.ds(0, B)], buf(k), ssem[k]).wait()

    def wait_gather(k):
        pltpu.make_async_copy(z128.at[pl.ds(0, B)], buf(k), gsem[k]).wait()

    fire_idx(0, 0)

    def macro(m, _):
        for k in range(2 * _SLOTS):
            b = m * (2 * _SLOTS) + k
            q = k % _SLOTS
            kn = (k + 1) % (2 * _SLOTS)
            qs = (k - 1) % _SLOTS
            ki = (k - 1) % (2 * _SLOTS)

            @pl.when(b + 1 < nb)
            def _():
                fire_idx(b + 1, kn)

            @pl.when(b < nb)
            def _():
                @pl.when(b >= _SLOTS)
                def _():
                    wait_scatter(q)
                wait_idx(k)
                pltpu.async_copy(table.at[sidx_v.at[k]], buf(q), gsem[q])

            bs = b - 1

            @pl.when(jnp.logical_and(bs >= 0, bs < nb))
            def _():
                wait_gather(qs)
                pltpu.async_copy(buf(qs), acc_sh.at[didx_v.at[ki]],
                                 ssem[qs], add=True)
        return 0

    lax.fori_loop(0, (nb + 2 * _SLOTS) // (2 * _SLOTS), macro, 0)
    for k in range(_SLOTS):
        wait_scatter(k)
    plsc.subcore_barrier()
    pltpu.sync_copy(acc_sh.at[pl.ds(r0, ROWS_T)],
                    out_hbm.at[cid, pl.ds(r0, ROWS_T)])


def _agg_call(table, src2d, dst2d, z128):
    body = functools.partial(_agg_body, nbr=0)
    return pl.kernel(
        body,
        out_type=jax.ShapeDtypeStruct((NC, ACC_N, D), jnp.float32),
        mesh=_mesh(),
        scratch_types=[
            pltpu.VMEM((2 * _SLOTS, B), jnp.int32),
            pltpu.VMEM((2 * _SLOTS, B), jnp.int32),
            pltpu.VMEM((_SLOTS * B, D), jnp.float32),
            pltpu.VMEM_SHARED((ACC_N, D), jnp.float32),
        ] + [pltpu.SemaphoreType.DMA] * (4 * _SLOTS),
    )(table, src2d, dst2d, z128)


_R = 1000


def _dis_body(dp_ref, o_ref):
    deg = 1.0 + jnp.sum(dp_ref[...], axis=0)
    o_ref[...] = lax.rsqrt(deg)


def _dis_call(dp):
    return pl.pallas_call(
        _dis_body,
        out_shape=jax.ShapeDtypeStruct((ACC_N // B, B), jnp.float32),
    )(dp)


def _t1_body(x_ref, fm_ref, w1_ref, dis_ref, o_ref):
    xm = x_ref[...] * jax.nn.sigmoid(fm_ref[...])
    o_ref[...] = jnp.dot(xm, w1_ref[...],
                         preferred_element_type=jnp.float32) * dis_ref[...]


def _t2_body(p_ref, y1_ref, dis_ref, b1_ref, w2_ref, o_ref):
    dis = dis_ref[...]
    s = p_ref[0] + p_ref[1]
    h = jax.nn.relu(dis * (s + y1_ref[...]) + b1_ref[...])
    o_ref[...] = jnp.dot(h, w2_ref[...],
                         preferred_element_type=jnp.float32) * dis


def _t3_body(p_ref, y2_ref, dis_ref, b2_ref, wc_ref, bc_ref, o_ref):
    dis = dis_ref[...]
    z = dis * (p_ref[0] + p_ref[1] + y2_ref[...]) + b2_ref[...]
    o_ref[...] = jnp.dot(z, wc_ref[...],
                         preferred_element_type=jnp.float32) + bc_ref[...]


def _row_specs():
    full = pl.BlockSpec((1, D), lambda i: (0, 0))
    mat = pl.BlockSpec((D, D), lambda i: (0, 0))
    rows = pl.BlockSpec((_R, D), lambda i: (i, 0))
    dis = pl.BlockSpec((_R, 1), lambda i: (i, 0))
    pair = pl.BlockSpec((2, _R, D), lambda i: (0, i, 0))
    return full, mat, rows, dis, pair


def _t1_call(x, fm, w1, dis):
    full, mat, rows, dis_s, _ = _row_specs()
    return pl.pallas_call(
        _t1_body,
        grid=(N // _R,),
        in_specs=[rows, full, mat, dis_s],
        out_specs=rows,
        out_shape=jax.ShapeDtypeStruct((N, D), jnp.float32),
    )(x, fm, w1, dis)


def _t2_call(p, y1, dis, b1, w2):
    full, mat, rows, dis_s, pair = _row_specs()
    return pl.pallas_call(
        _t2_body,
        grid=(N // _R,),
        in_specs=[pair, rows, dis_s, full, mat],
        out_specs=rows,
        out_shape=jax.ShapeDtypeStruct((N, D), jnp.float32),
    )(p, y1, dis, b1, w2)


def _t3_call(p, y2, dis, b2, wc, bc):
    full, mat, rows, dis_s, pair = _row_specs()
    return pl.pallas_call(
        _t3_body,
        grid=(N // _R,),
        in_specs=[pair, rows, dis_s, full, mat, full],
        out_specs=rows,
        out_shape=jax.ShapeDtypeStruct((N, D), jnp.float32),
    )(p, y2, dis, b2, wc, bc)


def kernel(x, edge_index, feature_mask, W1, b1, W2, b2, Wc, bc):
    src = edge_index[0]
    dst = edge_index[1]
    pad = E_PAD - E
    src_p = jnp.concatenate([src, jnp.zeros((pad,), jnp.int32)])
    dst_p = jnp.concatenate([dst, jnp.full((pad,), N, jnp.int32)])
    src2d = src_p.reshape(E_PAD // B, B)
    dst2d = dst_p.reshape(E_PAD // B, B)

    z128 = jnp.zeros((ACC_N, D), jnp.float32)

    fm = feature_mask.reshape(1, D)
    b1r = b1.reshape(1, D)
    b2r = b2.reshape(1, D)
    wc_pad = jnp.pad(Wc, ((0, 0), (0, D - Wc.shape[1])))
    bc_pad = jnp.pad(bc, (0, D - bc.shape[0])).reshape(1, D)

    dp = _deg_call(dst_p)
    dis = _dis_call(dp).reshape(ACC_N)[:N].reshape(N, 1)

    y1 = _t1_call(x, fm, W1, dis)
    p1 = _agg_call(y1, src2d, dst2d, z128)[:, :N]
    y2 = _t2_call(p1, y1, dis, b1r, W2)
    p2 = _agg_call(y2, src2d, dst2d, z128)[:, :N]
    out = _t3_call(p2, y2, dis, b2r, wc_pad, bc_pad)
    return out[:, :Wc.shape[1]]


# speedup vs baseline: 1.2062x; 1.0332x over previous
import functools

import jax
import jax.numpy as jnp
from jax import lax
from jax.experimental import pallas as pl
from jax.experimental.pallas import tpu as pltpu
from jax.experimental.pallas import tpu_sc as plsc

N = 10000
E = 320000
D = 128

NC = 2
NS = 16
NW = NC * NS
B = 128
NB_W = 79
E_PAD = NW * NB_W * B
ACC_N = 10112
ROWS_T = ACC_N // NS

_mesh = functools.partial(
    plsc.VectorSubcoreMesh, core_axis_name="c", subcore_axis_name="s")


def _deg_body(dst1d, out_hbm, idx_v, acc_v, *, nb):
    cid = lax.axis_index("c")
    sid = lax.axis_index("s")
    wid = sid * NC + cid
    nrow = ACC_N // B

    def zrow(i, _):
        def zcol(j, _):
            acc_v[i, pl.ds(j * 16, 16)] = jnp.zeros((16,), jnp.float32)
            return 0
        lax.fori_loop(0, B // 16, zcol, 0)
        return 0

    lax.fori_loop(0, nrow, zrow, 0)

    ones = jnp.ones((16,), jnp.float32)
    seven = jnp.full((16,), 7, jnp.int32)
    mask7 = jnp.full((16,), B - 1, jnp.int32)

    pltpu.sync_copy(dst1d.at[pl.ds(wid * nb * B, nb * B)], idx_v)

    def inner(j, _):
        idx = idx_v[pl.ds(j * 16, 16)]
        hi = lax.shift_right_logical(idx, seven)
        lo = lax.bitwise_and(idx, mask7)
        plsc.addupdate_scatter(acc_v, [hi, lo], ones)
        return 0

    lax.fori_loop(0, nb * B // 16, inner, 0)
    pltpu.sync_copy(acc_v, out_hbm.at[wid])


def _deg_call(dst1d):
    nb = dst1d.shape[0] // (NW * B)
    body = functools.partial(_deg_body, nb=nb)
    return pl.kernel(
        body,
        out_type=jax.ShapeDtypeStruct((NW, ACC_N // B, B), jnp.float32),
        mesh=_mesh(),
        scratch_types=[
            pltpu.VMEM((NB_W * B,), jnp.int32),
            pltpu.VMEM((ACC_N // B, B), jnp.float32),
        ],
        compiler_params=pltpu.CompilerParams(needs_layout_passes=False),
    )(dst1d)


NB0 = 132
NB1 = 2 * NB_W - NB0


def _agg_body(table, src1d, dst2d, z128, out_hbm,
              sidx_v, didx_v, bufs_v, acc_sh, gs0, gs1, ss0, ss1, *, nbr):
    cid = lax.axis_index("c")
    sid = lax.axis_index("s")
    r0 = sid * ROWS_T
    gsem = (gs0, gs1)
    ssem = (ss0, ss1)
    nb = jnp.where(cid == 0, NB0, NB1)
    base_row = sid * (NB0 + NB1) + cid * NB0
    pltpu.sync_copy(src1d.at[pl.ds(base_row * B, NB0 * B)], sidx_v)
    pltpu.sync_copy(z128.at[pl.ds(r0, ROWS_T)], acc_sh.at[pl.ds(r0, ROWS_T)])
    plsc.subcore_barrier()

    def buf(k):
        return bufs_v.at[pl.ds(k * B, B)]

    def wait_scatter(k):
        pltpu.make_async_copy(z128.at[pl.ds(0, B)], buf(k), ssem[k]).wait()

    def wait_gather(k):
        pltpu.make_async_copy(z128.at[pl.ds(0, B)], buf(k), gsem[k]).wait()
        pltpu.make_async_copy(dst2d.at[pl.ds(0, 1)],
                              didx_v.at[pl.ds(k, 1)], gsem[k]).wait()

    def macro(m, _):
        for k in range(2):
            b = m * 2 + k

            @pl.when(b < nb)
            def _():
                @pl.when(b >= 2)
                def _():
                    wait_scatter(k)
                pltpu.async_copy(dst2d.at[pl.ds(base_row + b, 1)],
                                 didx_v.at[pl.ds(k, 1)], gsem[k])
                pltpu.async_copy(table.at[sidx_v.at[pl.ds(b * B, B)]],
                                 buf(k), gsem[k])

            bs = b - 1
            ks = 1 - k

            @pl.when(jnp.logical_and(bs >= 0, bs < nb))
            def _():
                wait_gather(ks)
                pltpu.async_copy(buf(ks), acc_sh.at[didx_v.at[ks]],
                                 ssem[ks], add=True)
        return 0

    lax.fori_loop(0, (nb + 3) // 2, macro, 0)
    wait_scatter(0)
    wait_scatter(1)
    plsc.subcore_barrier()
    pltpu.sync_copy(acc_sh.at[pl.ds(r0, ROWS_T)],
                    out_hbm.at[cid, pl.ds(r0, ROWS_T)])


def _agg_call(table, src1d, dst2d, z128):
    body = functools.partial(_agg_body, nbr=0)
    return pl.kernel(
        body,
        out_type=jax.ShapeDtypeStruct((NC, ACC_N, D), jnp.float32),
        mesh=_mesh(),
        scratch_types=[
            pltpu.VMEM((NB0 * B,), jnp.int32),
            pltpu.VMEM((2, B), jnp.int32),
            pltpu.VMEM((2 * B, D), jnp.float32),
            pltpu.VMEM_SHARED((ACC_N, D), jnp.float32),
        ] + [pltpu.SemaphoreType.DMA] * 4,
    )(table, src1d, dst2d, z128)


_R = 1000


def _dis_body(dp_ref, o_ref):
    deg = 1.0 + jnp.sum(dp_ref[...], axis=0)
    o_ref[...] = lax.rsqrt(deg)


def _dis_call(dp):
    return pl.pallas_call(
        _dis_body,
        out_shape=jax.ShapeDtypeStruct((ACC_N // B, B), jnp.float32),
    )(dp)


def _t1_body(x_ref, fm_ref, w1_ref, dis_ref, o_ref):
    xm = x_ref[...] * jax.nn.sigmoid(fm_ref[...])
    o_ref[...] = jnp.dot(xm, w1_ref[...],
                         preferred_element_type=jnp.float32) * dis_ref[...]


def _t2_body(p_ref, y1_ref, dis_ref, b1_ref, w2_ref, o_ref):
    dis = dis_ref[...]
    s = p_ref[0] + p_ref[1]
    h = jax.nn.relu(dis * (s + y1_ref[...]) + b1_ref[...])
    o_ref[...] = jnp.dot(h, w2_ref[...],
                         preferred_element_type=jnp.float32) * dis


def _t3_body(p_ref, y2_ref, dis_ref, b2_ref, wc_ref, bc_ref, o_ref):
    dis = dis_ref[...]
    z = dis * (p_ref[0] + p_ref[1] + y2_ref[...]) + b2_ref[...]
    o_ref[...] = jnp.dot(z, wc_ref[...],
                         preferred_element_type=jnp.float32) + bc_ref[...]


def _row_specs():
    full = pl.BlockSpec((1, D), lambda i: (0, 0))
    mat = pl.BlockSpec((D, D), lambda i: (0, 0))
    rows = pl.BlockSpec((_R, D), lambda i: (i, 0))
    dis = pl.BlockSpec((_R, 1), lambda i: (i, 0))
    pair = pl.BlockSpec((2, _R, D), lambda i: (0, i, 0))
    return full, mat, rows, dis, pair


def _t1_call(x, fm, w1, dis):
    full, mat, rows, dis_s, _ = _row_specs()
    return pl.pallas_call(
        _t1_body,
        grid=(N // _R,),
        in_specs=[rows, full, mat, dis_s],
        out_specs=rows,
        out_shape=jax.ShapeDtypeStruct((N, D), jnp.float32),
    )(x, fm, w1, dis)


def _t2_call(p, y1, dis, b1, w2):
    full, mat, rows, dis_s, pair = _row_specs()
    return pl.pallas_call(
        _t2_body,
        grid=(N // _R,),
        in_specs=[pair, rows, dis_s, full, mat],
        out_specs=rows,
        out_shape=jax.ShapeDtypeStruct((N, D), jnp.float32),
    )(p, y1, dis, b1, w2)


def _t3_call(p, y2, dis, b2, wc, bc):
    full, mat, rows, dis_s, pair = _row_specs()
    return pl.pallas_call(
        _t3_body,
        grid=(N // _R,),
        in_specs=[pair, rows, dis_s, full, mat, full],
        out_specs=rows,
        out_shape=jax.ShapeDtypeStruct((N, D), jnp.float32),
    )(p, y2, dis, b2, wc, bc)


def kernel(x, edge_index, feature_mask, W1, b1, W2, b2, Wc, bc):
    src = edge_index[0]
    dst = edge_index[1]
    pad = E_PAD - E
    src_len = (15 * (NB0 + NB1) + NB0 + NB0) * B
    src1d = jnp.concatenate([src, jnp.zeros((src_len - E,), jnp.int32)])
    dst_p = jnp.concatenate([dst, jnp.full((pad,), N, jnp.int32)])
    dst2d = dst_p.reshape(E_PAD // B, B)

    z128 = jnp.zeros((ACC_N, D), jnp.float32)

    fm = feature_mask.reshape(1, D)
    b1r = b1.reshape(1, D)
    b2r = b2.reshape(1, D)
    wc_pad = jnp.pad(Wc, ((0, 0), (0, D - Wc.shape[1])))
    bc_pad = jnp.pad(bc, (0, D - bc.shape[0])).reshape(1, D)

    dp = _deg_call(dst_p)
    dis = _dis_call(dp).reshape(ACC_N)[:N].reshape(N, 1)

    y1 = _t1_call(x, fm, W1, dis)
    p1 = _agg_call(y1, src1d, dst2d, z128)[:, :N]
    y2 = _t2_call(p1, y1, dis, b1r, W2)
    p2 = _agg_call(y2, src1d, dst2d, z128)[:, :N]
    out = _t3_call(p2, y2, dis, b2r, wc_pad, bc_pad)
    return out[:, :Wc.shape[1]]
